# Initial kernel scaffold; baseline (speedup 1.0000x reference)
import jax
import jax.numpy as jnp
from jax.experimental import pallas as pl


def kernel(points, params):
    raise NotImplementedError("write your pallas kernel here")


# speedup vs baseline: 3.4400x; 3.4400x over previous
import functools

import jax
import jax.numpy as jnp
from jax.experimental import pallas as pl
from jax.experimental.pallas import tpu as pltpu

_NPTS = 16384
_NKP = 2048
_STRIDES = (1, 2, 4, 8)
_CHS = (1, 16, 32, 64)
_RADII = ((0.4, 0.8), (0.8, 1.2), (1.2, 2.4), (2.4, 4.8))
_MLP = (16, 32, 64, 64)
_NS = 16
_NPROP = 128
_GRID = 4
_XMIN, _YMIN = 0.0, -40.0
_RES = 0.4
_BEVH, _BEVW = 176, 200
_CELLS = _BEVH * _BEVW

_CP = pltpu.CompilerParams(vmem_limit_bytes=100 * 1024 * 1024)
_DN = (((1,), (0,)), ((), ()))


def _dot_bf(a, b):
    return jax.lax.dot_general(a.astype(jnp.bfloat16), b.astype(jnp.bfloat16),
                               _DN, preferred_element_type=jnp.float32)


def _dot_hi(a, b):
    return jax.lax.dot_general(a, b, _DN,
                               precision=jax.lax.Precision.HIGHEST,
                               preferred_element_type=jnp.float32)


def _fps_body(xs_ref, ys_ref, zs_ref, idx_ref):
    xs = xs_ref[...]
    ys = ys_ref[...]
    zs = zs_ref[...]
    iota = (jax.lax.broadcasted_iota(jnp.int32, (128, 128), 0) * 128
            + jax.lax.broadcasted_iota(jnp.int32, (128, 128), 1))
    idx_ref[0:1, :] = jnp.zeros((1, 1), jnp.int32)

    def body(i, carry):
        dists, p = carry
        sel = iota == p
        lx = jnp.sum(jnp.where(sel, xs, 0.0))
        ly = jnp.sum(jnp.where(sel, ys, 0.0))
        lz = jnp.sum(jnp.where(sel, zs, 0.0))
        dx = xs - lx
        dy = ys - ly
        dz = zs - lz
        d = dx * dx + dy * dy + dz * dz
        dists = jnp.minimum(dists, d)
        m = jnp.max(dists)
        pn = jnp.min(jnp.where(dists == m, iota, jnp.int32(2**30)))
        idx_ref[pl.ds(i, 1), :] = pn.reshape(1, 1)
        return dists, pn

    dists0 = jnp.full((128, 128), 1e10, jnp.float32)
    jax.lax.fori_loop(1, _NKP, body, (dists0, jnp.int32(0)))


def _fps_call(points):
    xs = points[:, 0].reshape(128, 128)
    ys = points[:, 1].reshape(128, 128)
    zs = points[:, 2].reshape(128, 128)
    idx2d = pl.pallas_call(
        _fps_body,
        out_shape=jax.ShapeDtypeStruct((_NKP, 1), jnp.int32),
        compiler_params=_CP,
    )(xs, ys, zs)
    return idx2d[:, 0]


def _stride_body(i_stride, tk, *refs):
    c = _CHS[i_stride]
    f = _MLP[i_stride]
    r0, r1 = _RADII[i_stride]
    ns = _NPTS // _STRIDES[i_stride]
    if i_stride == 0:
        (kp_ref, pts_ref, ptsT_ref,
         w10a_ref, w10b_ref, b10_ref, w20_ref, b20_ref,
         w11a_ref, w11b_ref, b11_ref, w21_ref, b21_ref,
         out0_ref, out1_ref) = refs
    else:
        (kp_ref, pts_ref, ptsT_ref, wp_ref, bp_ref,
         w10a_ref, w10b_ref, b10_ref, w20_ref, b20_ref,
         w11a_ref, w11b_ref, b11_ref, w21_ref, b21_ref,
         out0_ref, out1_ref) = refs

    kp3 = kp_ref[...][:, :3]
    ptsT = ptsT_ref[...]
    vxT = ptsT[:3, :]
    s_vx = jnp.sum(vxT * vxT, axis=0, keepdims=True)
    s_kp = jnp.sum(kp3 * kp3, axis=1, keepdims=True)
    cross = _dot_bf(kp3, vxT)
    d2_0 = s_kp + s_vx - 2.0 * cross

    pts = pts_ref[...]
    vx3 = pts[:, :3]
    if i_stride == 0:
        feat = pts[:, 3:4]
    else:
        feat = jax.nn.relu(_dot_bf(pts, wp_ref[...]) + bp_ref[...])

    w10a = w10a_ref[...]
    w10b = w10b_ref[...]
    b10 = b10_ref[...]
    w20 = w20_ref[...]
    b20 = b20_ref[...]
    w11a = w11a_ref[...]
    w11b = w11b_ref[...]
    b11 = b11_ref[...]
    w21 = w21_ref[...]
    b21 = b21_ref[...]

    lane = jax.lax.broadcasted_iota(jnp.int32, (tk, ns), 1)

    def it(_, carry):
        d2, p0, p1 = carry
        m = jnp.min(d2, axis=1, keepdims=True)
        sel = jnp.min(jnp.where(d2 == m, lane, ns), axis=1, keepdims=True)
        oh = (lane == sel).astype(jnp.float32)
        gvx = _dot_hi(oh, vx3) - kp3
        gf = _dot_hi(oh, feat)
        h0 = jax.nn.relu(_dot_bf(gvx, w10a) + _dot_bf(gf, w10b) + b10)
        h0 = jax.nn.relu(_dot_bf(h0, w20) + b20)
        h1 = jax.nn.relu(_dot_bf(gvx, w11a) + _dot_bf(gf, w11b) + b11)
        h1 = jax.nn.relu(_dot_bf(h1, w21) + b21)
        p0 = jnp.maximum(p0, jnp.where(m < r0 * r0, h0, -1e9))
        p1 = jnp.maximum(p1, jnp.where(m < r1 * r1, h1, -1e9))
        d2 = jnp.where(lane == sel, jnp.float32(jnp.inf), d2)
        return d2, p0, p1

    init = (d2_0,
            jnp.full((tk, f), -1e9, jnp.float32),
            jnp.full((tk, f), -1e9, jnp.float32))
    _, p0, p1 = jax.lax.fori_loop(0, _NS, it, init)
    out0_ref[...] = jnp.where(p0 > -1e8, p0, 0.0)
    out1_ref[...] = jnp.where(p1 > -1e8, p1, 0.0)


def _stride_call(i_stride, kp4, pts_s, params):
    c = _CHS[i_stride]
    f = _MLP[i_stride]
    ns = _NPTS // _STRIDES[i_stride]
    tk = 64 if i_stride == 0 else 128
    grid = (_NKP // tk,)

    full = lambda shp: pl.BlockSpec(shp, lambda i: (0,) * len(shp))
    tile = lambda shp: pl.BlockSpec(shp, lambda i: (i,) + (0,) * (len(shp) - 1))

    args = [kp4, pts_s, pts_s.T]
    specs = [tile((tk, 4)), full((ns, 4)), full((4, ns))]
    if i_stride > 0:
        wp, bp = params["stride_proj"][i_stride - 1]
        args += [wp, bp.reshape(1, c)]
        specs += [full((4, c)), full((1, c))]
    for j in range(2):
        w1, b1, w2, b2 = params["pnet"][i_stride][j]
        args += [w1[:3], w1[3:], b1.reshape(1, f), w2, b2.reshape(1, f)]
        specs += [full((3, f)), full((c, f)), full((1, f)),
                  full((f, f)), full((1, f))]

    out0, out1 = pl.pallas_call(
        functools.partial(_stride_body, i_stride, tk),
        grid=grid,
        in_specs=specs,
        out_specs=[tile((tk, f)), tile((tk, f))],
        out_shape=[jax.ShapeDtypeStruct((_NKP, f), jnp.float32)] * 2,
        compiler_params=_CP,
    )(*args)
    return out0, out1


def _bev_body(pts8_ref, wp_ref, bp_ref, wb_ref, bb_ref,
              flat_ref, x0_ref, y0_ref, tx_ref, ty_ref,
              out_ref, acc_ref, upd_ref):
    pts8 = pts8_ref[...]
    feats8 = jax.nn.relu(_dot_bf(pts8, wp_ref[...]) + bp_ref[...])
    bevf = jax.nn.relu(_dot_bf(feats8, wb_ref[...]) + bb_ref[...])
    upd_ref[...] = jnp.concatenate(
        [bevf, jnp.ones((_NKP, 1), jnp.float32),
         jnp.zeros((_NKP, 63), jnp.float32)], axis=1)
    acc_ref[...] = jnp.zeros((_CELLS, 128), jnp.float32)

    def scatter(p, _):
        r = flat_ref[p]
        acc_ref[pl.ds(r, 1), :] = (acc_ref[pl.ds(r, 1), :]
                                   + upd_ref[pl.ds(p, 1), :])
        return 0

    jax.lax.fori_loop(0, _NKP, scatter, 0)

    def gather(q, _):
        x0 = x0_ref[q]
        y0 = y0_ref[q]
        tx = tx_ref[q]
        ty = ty_ref[q]
        r00 = x0 * _BEVW + y0

        def corner(r):
            row = acc_ref[pl.ds(r, 1), :]
            return row[:, 0:64] / jnp.maximum(row[:, 64:65], 1.0)

        v = (corner(r00) * ((1.0 - tx) * (1.0 - ty))
             + corner(r00 + _BEVW) * (tx * (1.0 - ty))
             + corner(r00 + 1) * ((1.0 - tx) * ty)
             + corner(r00 + _BEVW + 1) * (tx * ty))
        out_ref[pl.ds(q, 1), :] = v
        return 0

    jax.lax.fori_loop(0, _NKP, gather, 0)


def _bev_call(pts8, kp4, params):
    wp, bp = params["stride_proj"][2]
    wb, bb = params["bev"]
    ix = jnp.clip(jnp.floor((pts8[:, 0] - _XMIN) / _RES).astype(jnp.int32),
                  0, _BEVH - 1)
    iy = jnp.clip(jnp.floor((pts8[:, 1] - _YMIN) / _RES).astype(jnp.int32),
                  0, _BEVW - 1)
    flat = ix * _BEVW + iy
    fx = (kp4[:, 0] - _XMIN) / _RES - 0.5
    fy = (kp4[:, 1] - _YMIN) / _RES - 0.5
    x0 = jnp.clip(jnp.floor(fx).astype(jnp.int32), 0, _BEVH - 2)
    y0 = jnp.clip(jnp.floor(fy).astype(jnp.int32), 0, _BEVW - 2)
    tx = jnp.clip(fx - x0, 0.0, 1.0)
    ty = jnp.clip(fy - y0, 0.0, 1.0)

    vm = lambda shp: pl.BlockSpec(shp, lambda: (0,) * len(shp))
    sm = pl.BlockSpec(memory_space=pltpu.SMEM)
    return pl.pallas_call(
        _bev_body,
        in_specs=[vm((_NKP, 4)), vm((4, 64)), vm((1, 64)),
                  vm((64, 64)), vm((1, 64)), sm, sm, sm, sm, sm],
        out_specs=vm((_NKP, 64)),
        out_shape=jax.ShapeDtypeStruct((_NKP, 64), jnp.float32),
        scratch_shapes=[pltpu.VMEM((_CELLS, 128), jnp.float32),
                        pltpu.VMEM((_NKP, 128), jnp.float32)],
        compiler_params=_CP,
    )(pts8, wp, bp.reshape(1, 64), wb, bb.reshape(1, 64),
      flat, x0, y0, tx, ty)


def _head_body(featT_ref, feats_ref, kp4_ref, wcT_ref, wbox_ref,
               ctr_ref, dim_ref, p_ref):
    scores0 = _dot_bf(wcT_ref[...], featT_ref[...])
    lane = jax.lax.broadcasted_iota(jnp.int32, (1, _NKP), 1)

    def it(k, s):
        m = jnp.max(s)
        sel = jnp.min(jnp.where(s == m, lane, jnp.int32(2**30)))
        p_ref[pl.ds(k, 1), :] = (lane == sel).astype(jnp.float32)
        return jnp.where(lane == sel, -jnp.inf, s)

    jax.lax.fori_loop(0, _NPROP, it, scores0)
    psel = p_ref[...]
    deltas = _dot_bf(feats_ref[...], wbox_ref[...])
    gkp = _dot_hi(psel, kp4_ref[...])
    gd = _dot_hi(psel, deltas)
    ctr_ref[...] = gkp[:, :3] + gd[:, :3]
    dim_ref[...] = jax.nn.softplus(gd[:, 3:6]) + 1.0


def _head_call(features, kp4, params):
    wc, wbox = params["prop"]
    vm = lambda shp: pl.BlockSpec(shp, lambda: (0,) * len(shp))
    ctr, dim = pl.pallas_call(
        _head_body,
        in_specs=[vm((416, _NKP)), vm((_NKP, 416)), vm((_NKP, 4)),
                  vm((1, 416)), vm((416, 7))],
        out_specs=[vm((_NPROP, 3)), vm((_NPROP, 3))],
        out_shape=[jax.ShapeDtypeStruct((_NPROP, 3), jnp.float32)] * 2,
        scratch_shapes=[pltpu.VMEM((_NPROP, _NKP), jnp.float32)],
        compiler_params=_CP,
    )(features.T, features, kp4, wc.T, wbox)
    return ctr, dim


def _roi_body(gp_ref, kpT_ref, kp3_ref, feats_ref, w1a_ref, w1b_ref, b1_ref,
              w2_ref, b2_ref, out_ref):
    tg = gp_ref.shape[0]
    gp3 = gp_ref[...][:, :3]
    kpT = kpT_ref[...]
    kp3T = kpT[:3, :]
    s_kp = jnp.sum(kp3T * kp3T, axis=0, keepdims=True)
    s_gp = jnp.sum(gp3 * gp3, axis=1, keepdims=True)
    cross = _dot_bf(gp3, kp3T)
    d2_0 = s_gp + s_kp - 2.0 * cross

    kp3m = kp3_ref[...]
    feats = feats_ref[...]
    w1a = w1a_ref[...]
    w1b = w1b_ref[...]
    b1 = b1_ref[...]
    w2 = w2_ref[...]
    b2 = b2_ref[...]
    lane = jax.lax.broadcasted_iota(jnp.int32, (tg, _NKP), 1)

    def it(_, carry):
        d2, p = carry
        m = jnp.min(d2, axis=1, keepdims=True)
        sel = jnp.min(jnp.where(d2 == m, lane, _NKP), axis=1, keepdims=True)
        oh = (lane == sel).astype(jnp.float32)
        gk = _dot_hi(oh, kp3m) - gp3
        gf = _dot_bf(oh, feats)
        h = jax.nn.relu(_dot_bf(gk, w1a) + _dot_bf(gf, w1b) + b1)
        h = jax.nn.relu(_dot_bf(h, w2) + b2)
        p = jnp.maximum(p, h)
        d2 = jnp.where(lane == sel, jnp.float32(jnp.inf), d2)
        return d2, p

    init = (d2_0, jnp.zeros((tg, 64), jnp.float32))
    _, p = jax.lax.fori_loop(0, _NS, it, init)
    out_ref[...] = p


def _roi_call(gp4, kp4, features, params):
    w1, b1, w2, b2, _, _ = params["roi"]
    ng = _NPROP * _GRID**3
    tg = 256
    full = lambda shp: pl.BlockSpec(shp, lambda i: (0,) * len(shp))
    tile = lambda shp: pl.BlockSpec(shp, lambda i: (i,) + (0,) * (len(shp) - 1))
    return pl.pallas_call(
        _roi_body,
        grid=(ng // tg,),
        in_specs=[tile((tg, 4)), full((4, _NKP)), full((_NKP, 3)),
                  full((_NKP, 416)), full((3, 64)), full((416, 64)),
                  full((1, 64)), full((64, 64)), full((1, 64))],
        out_specs=tile((tg, 64)),
        out_shape=jax.ShapeDtypeStruct((ng, 64), jnp.float32),
        compiler_params=_CP,
    )(gp4, kp4.T, kp4[:, :3], features, w1[:3], w1[3:],
      b1.reshape(1, 64), w2, b2.reshape(1, 64))


def _final_body(h_ref, wfc_ref, bfc_ref, wr1_ref, br1_ref,
                wrc_ref, brc_ref, wrr_ref, brr_ref, out_ref):
    pooled = jax.nn.relu(_dot_bf(h_ref[...], wfc_ref[...]) + bfc_ref[...])
    hh = jax.nn.relu(_dot_bf(pooled, wr1_ref[...]) + br1_ref[...])
    oc = _dot_bf(hh, wrc_ref[...]) + brc_ref[...]
    orr = _dot_bf(hh, wrr_ref[...]) + brr_ref[...]
    out_ref[...] = jnp.concatenate([oc, orr], axis=1)


def _final_call(hflat, params):
    wfc, bfc = params["roi"][4], params["roi"][5]
    wr1, br1, wrc, brc, wrr, brr = params["ref"]
    vm = lambda shp: pl.BlockSpec(shp, lambda: (0,) * len(shp))
    return pl.pallas_call(
        _final_body,
        in_specs=[vm((_NPROP, 4096)), vm((4096, 256)), vm((1, 256)),
                  vm((256, 256)), vm((1, 256)), vm((256, 1)), vm((1, 1)),
                  vm((256, 7)), vm((1, 7))],
        out_specs=vm((_NPROP, 8)),
        out_shape=jax.ShapeDtypeStruct((_NPROP, 8), jnp.float32),
        compiler_params=_CP,
    )(hflat, wfc, bfc.reshape(1, 256), wr1, br1.reshape(1, 256),
      wrc, brc.reshape(1, 1), wrr, brr.reshape(1, 7))


def kernel(points, params):
    kp_idx = _fps_call(points)
    kp4 = points[kp_idx]

    pnet_feats = []
    for i, s in enumerate(_STRIDES):
        p0, p1 = _stride_call(i, kp4, points[::s], params)
        pnet_feats += [p0, p1]

    bev_out = _bev_call(points[::8], kp4, params)
    features = jnp.concatenate(pnet_feats + [bev_out], axis=1)

    ctr, dim = _head_call(features, kp4, params)

    lin = jnp.linspace(-0.5, 0.5, _GRID)
    gx, gy, gz = jnp.meshgrid(lin, lin, lin, indexing="ij")
    offs = jnp.stack([gx.ravel(), gy.ravel(), gz.ravel()], axis=1)
    gp = (ctr[:, None, :] + offs[None, :, :] * dim[:, None, :]).reshape(-1, 3)
    gp4 = jnp.concatenate(
        [gp, jnp.zeros((gp.shape[0], 1), jnp.float32)], axis=1)

    hmax = _roi_call(gp4, kp4, features, params)
    hflat = hmax.reshape(_NPROP, _GRID**3 * 64)
    return _final_call(hflat, params)
